# baseline (speedup 1.0000x reference)
import jax
import jax.numpy as jnp
from jax import lax
from jax.experimental import pallas as pl
from jax.experimental.pallas import tpu as pltpu
from jax.experimental.pallas import tpu_sc as plsc

_ROWS, _COLS = 64, 32768
_NC, _NS, _L = 2, 16, 16
_NW = _NC * _NS
_SC_ROWS = 8
_TC_ROWS = _ROWS - _SC_ROWS
_WPR = _NW // _SC_ROWS
_WCOLS = _COLS // _WPR
_UNROLL = 4
_MESH = plsc.VectorSubcoreMesh(core_axis_name="c", subcore_axis_name="s")


def _sc_scan(x_hbm, part_hbm, buf, mvec_ref, svec_ref):
    wid = lax.axis_index("c") * _NS + lax.axis_index("s")
    row = lax.shift_right_logical(wid, 2)
    col0 = lax.mul(lax.rem(wid, _WPR), _WCOLS)
    pltpu.sync_copy(x_hbm.at[row, pl.ds(col0, _WCOLS)], buf)

    neg = jnp.full((_L,), -jnp.inf, jnp.float32)
    carry0 = (neg,) * (2 * _UNROLL)

    def body(i, carry):
        ms, ss = list(carry[:_UNROLL]), list(carry[_UNROLL:])
        base = i * (_UNROLL * _L)
        for j in range(_UNROLL):
            v = buf[pl.ds(base + j * _L, _L)]
            ss[j] = jnp.maximum(ss[j], jnp.minimum(ms[j], v))
            ms[j] = jnp.maximum(ms[j], v)
        return tuple(ms) + tuple(ss)

    carry = lax.fori_loop(0, _WCOLS // (_UNROLL * _L), body, carry0)
    ms, ss = list(carry[:_UNROLL]), list(carry[_UNROLL:])
    n = _UNROLL
    while n > 1:
        half = n // 2
        for j in range(half):
            m_a, s_a = ms[j], ss[j]
            m_b, s_b = ms[j + half], ss[j + half]
            ss[j] = jnp.maximum(jnp.minimum(m_a, m_b), jnp.maximum(s_a, s_b))
            ms[j] = jnp.maximum(m_a, m_b)
        n = half
    mvec_ref[...] = ms[0]
    svec_ref[...] = ss[0]
    pltpu.sync_copy(mvec_ref, part_hbm.at[wid])
    pltpu.sync_copy(svec_ref, part_hbm.at[_NW + wid])


_sc_partials = pl.kernel(
    _sc_scan,
    out_type=jax.ShapeDtypeStruct((2 * _NW, _L), jnp.float32),
    mesh=_MESH,
    scratch_types=[
        pltpu.VMEM((_WCOLS,), jnp.float32),
        pltpu.VMEM((_L,), jnp.float32),
        pltpu.VMEM((_L,), jnp.float32),
    ],
)

_TC_BLOCK = 8


def _tc_scan_kernel(x_ref, max_ref, cnt_ref, m_ref, s_ref):
    i = pl.program_id(0)
    blk = x_ref[...]

    @pl.when(i == 0)
    def _init():
        m_ref[...] = blk
        s_ref[...] = jnp.full_like(blk, -jnp.inf)

    @pl.when(i > 0)
    def _acc():
        m = m_ref[...]
        s_ref[...] = jnp.maximum(s_ref[...], jnp.minimum(m, blk))
        m_ref[...] = jnp.maximum(m, blk)

    @pl.when(i == pl.num_programs(0) - 1)
    def _emit():
        m = m_ref[...]
        big = jnp.max(m)
        cnt = jnp.sum((m == big).astype(jnp.int32)) + jnp.sum(
            (s_ref[...] == big).astype(jnp.int32)
        )
        max_ref[0, 0] = big
        cnt_ref[0, 0] = cnt


def _tc_scan(x):
    return pl.pallas_call(
        _tc_scan_kernel,
        grid=(_TC_ROWS // _TC_BLOCK,),
        in_specs=[
            pl.BlockSpec(
                (_TC_BLOCK, _COLS), lambda i: (i + _SC_ROWS // _TC_BLOCK, 0)
            ),
        ],
        out_specs=(
            pl.BlockSpec(memory_space=pltpu.SMEM),
            pl.BlockSpec(memory_space=pltpu.SMEM),
        ),
        out_shape=(
            jax.ShapeDtypeStruct((1, 1), jnp.float32),
            jax.ShapeDtypeStruct((1, 1), jnp.int32),
        ),
        scratch_shapes=[
            pltpu.VMEM((_TC_BLOCK, _COLS), jnp.float32),
            pltpu.VMEM((_TC_BLOCK, _COLS), jnp.float32),
        ],
    )(x)


def _combine_kernel(parts_ref, tmax_ref, tcnt_ref, out_ref):
    parts = parts_ref[...]
    tmax = tmax_ref[0, 0]
    m = jnp.maximum(jnp.max(parts), tmax)
    total = jnp.sum((parts == m).astype(jnp.int32)) + jnp.where(
        tmax == m, tcnt_ref[0, 0], 0
    )
    out_ref[0, 0] = (total > 1).astype(jnp.int32)


def kernel(x):
    parts = _sc_partials(x)
    tmax, tcnt = _tc_scan(x)
    out = pl.pallas_call(
        _combine_kernel,
        in_specs=[
            pl.BlockSpec(),
            pl.BlockSpec(memory_space=pltpu.SMEM),
            pl.BlockSpec(memory_space=pltpu.SMEM),
        ],
        out_specs=pl.BlockSpec(memory_space=pltpu.SMEM),
        out_shape=jax.ShapeDtypeStruct((1, 1), jnp.int32),
    )(parts, tmax, tcnt)
    return out.reshape(()).astype(jnp.bool_)


# speedup vs baseline: 1.0484x; 1.0484x over previous
import jax
import jax.numpy as jnp
from jax import lax
from jax.experimental import pallas as pl
from jax.experimental.pallas import tpu as pltpu
from jax.experimental.pallas import tpu_sc as plsc

_ROWS, _COLS = 64, 32768
_NC, _NS, _L = 2, 16, 16
_NW = _NC * _NS
_SC_ROWS = 16
_TC_ROWS = _ROWS - _SC_ROWS
_WCOLS = _COLS // 2
_UNROLL = 4
_MESH = plsc.VectorSubcoreMesh(core_axis_name="c", subcore_axis_name="s")


def _sc_scan(x_hbm, part_hbm, buf, mvec_ref, svec_ref):
    wid = lax.axis_index("c") * _NS + lax.axis_index("s")
    row = lax.shift_right_logical(wid, 1)
    col0 = lax.mul(lax.rem(wid, 2), _WCOLS)
    pltpu.sync_copy(x_hbm.at[row, pl.ds(col0, _WCOLS)], buf)

    neg = jnp.full((_L,), -jnp.inf, jnp.float32)
    carry0 = (neg,) * (2 * _UNROLL)

    def body(i, carry):
        ms, ss = list(carry[:_UNROLL]), list(carry[_UNROLL:])
        base = i * (_UNROLL * _L)
        for j in range(_UNROLL):
            v = buf[pl.ds(base + j * _L, _L)]
            ss[j] = jnp.maximum(ss[j], jnp.minimum(ms[j], v))
            ms[j] = jnp.maximum(ms[j], v)
        return tuple(ms) + tuple(ss)

    carry = lax.fori_loop(0, _WCOLS // (_UNROLL * _L), body, carry0)
    ms, ss = list(carry[:_UNROLL]), list(carry[_UNROLL:])
    n = _UNROLL
    while n > 1:
        half = n // 2
        for j in range(half):
            m_a, s_a = ms[j], ss[j]
            m_b, s_b = ms[j + half], ss[j + half]
            ss[j] = jnp.maximum(jnp.minimum(m_a, m_b), jnp.maximum(s_a, s_b))
            ms[j] = jnp.maximum(m_a, m_b)
        n = half
    mvec_ref[...] = ms[0]
    svec_ref[...] = ss[0]
    pltpu.sync_copy(mvec_ref, part_hbm.at[wid])
    pltpu.sync_copy(svec_ref, part_hbm.at[_NW + wid])


_sc_partials = pl.kernel(
    _sc_scan,
    out_type=jax.ShapeDtypeStruct((2 * _NW, _L), jnp.float32),
    mesh=_MESH,
    scratch_types=[
        pltpu.VMEM((_WCOLS,), jnp.float32),
        pltpu.VMEM((_L,), jnp.float32),
        pltpu.VMEM((_L,), jnp.float32),
    ],
)

_TC_BLOCK = 16


def _tc_scan_kernel(x_ref, max_ref, cnt_ref, m_ref, c_ref):
    i = pl.program_id(0)
    blk = x_ref[...]
    bm = jnp.max(blk)
    bc = jnp.sum((blk == bm).astype(jnp.int32))

    @pl.when(i == 0)
    def _init():
        m_ref[0] = bm
        c_ref[0] = bc

    @pl.when(i > 0)
    def _acc():
        m = m_ref[0]
        c = c_ref[0]
        m_ref[0] = jnp.maximum(m, bm)
        c_ref[0] = jnp.where(bm > m, bc, jnp.where(bm == m, c + bc, c))

    @pl.when(i == pl.num_programs(0) - 1)
    def _emit():
        max_ref[0, 0] = m_ref[0]
        cnt_ref[0, 0] = c_ref[0]


def _tc_scan(x):
    return pl.pallas_call(
        _tc_scan_kernel,
        grid=(_TC_ROWS // _TC_BLOCK,),
        in_specs=[
            pl.BlockSpec(
                (_TC_BLOCK, _COLS), lambda i: (i + _SC_ROWS // _TC_BLOCK, 0)
            ),
        ],
        out_specs=(
            pl.BlockSpec(memory_space=pltpu.SMEM),
            pl.BlockSpec(memory_space=pltpu.SMEM),
        ),
        out_shape=(
            jax.ShapeDtypeStruct((1, 1), jnp.float32),
            jax.ShapeDtypeStruct((1, 1), jnp.int32),
        ),
        scratch_shapes=[
            pltpu.SMEM((1,), jnp.float32),
            pltpu.SMEM((1,), jnp.int32),
        ],
    )(x)


def _combine_kernel(parts_ref, tmax_ref, tcnt_ref, out_ref):
    parts = parts_ref[...]
    tmax = tmax_ref[0, 0]
    m = jnp.maximum(jnp.max(parts), tmax)
    total = jnp.sum((parts == m).astype(jnp.int32)) + jnp.where(
        tmax == m, tcnt_ref[0, 0], 0
    )
    out_ref[0, 0] = (total > 1).astype(jnp.int32)


def kernel(x):
    parts = _sc_partials(x)
    tmax, tcnt = _tc_scan(x)
    out = pl.pallas_call(
        _combine_kernel,
        in_specs=[
            pl.BlockSpec(),
            pl.BlockSpec(memory_space=pltpu.SMEM),
            pl.BlockSpec(memory_space=pltpu.SMEM),
        ],
        out_specs=pl.BlockSpec(memory_space=pltpu.SMEM),
        out_shape=jax.ShapeDtypeStruct((1, 1), jnp.int32),
    )(parts, tmax, tcnt)
    return out.reshape(()).astype(jnp.bool_)
